# baseline (speedup 1.0000x reference)
import jax
import jax.numpy as jnp
from jax.experimental import pallas as pl
from jax.experimental.pallas import tpu as pltpu

_ROWS = 8192
_COLS = 128
_N_STRIPES = 4
_STRIPE = _ROWS // _N_STRIPES


def _dma_copy(t_ref, o_ref, *sems):
    copies = [
        pltpu.make_async_copy(
            t_ref.at[pl.ds(i * _STRIPE, _STRIPE), :],
            o_ref.at[pl.ds(i * _STRIPE, _STRIPE), :],
            sems[i],
        )
        for i in range(_N_STRIPES)
    ]
    for c in copies:
        c.start()
    for c in copies:
        c.wait()


def kernel(x, pos_table):
    del x
    return pl.pallas_call(
        _dma_copy,
        out_shape=jax.ShapeDtypeStruct((_ROWS, _COLS), pos_table.dtype),
        in_specs=[pl.BlockSpec(memory_space=pl.ANY)],
        out_specs=pl.BlockSpec(memory_space=pl.ANY),
        scratch_shapes=[pltpu.SemaphoreType.DMA] * _N_STRIPES,
    )(pos_table)


# speedup vs baseline: 11.9147x; 11.9147x over previous
import jax
import jax.numpy as jnp
from jax.experimental import pallas as pl

_ROWS = 8192
_COLS = 128
_BLOCK_ROWS = 512


def _copy_block(t_ref, o_ref):
    o_ref[...] = t_ref[...]


def kernel(x, pos_table):
    del x
    n_blocks = _ROWS // _BLOCK_ROWS
    return pl.pallas_call(
        _copy_block,
        out_shape=jax.ShapeDtypeStruct((_ROWS, _COLS), pos_table.dtype),
        grid=(n_blocks,),
        in_specs=[pl.BlockSpec((_BLOCK_ROWS, _COLS), lambda i: (i, 0))],
        out_specs=pl.BlockSpec((_BLOCK_ROWS, _COLS), lambda i: (i, 0)),
    )(pos_table)


# speedup vs baseline: 26.0721x; 2.1882x over previous
import jax
import jax.numpy as jnp
from jax.experimental import pallas as pl

_ROWS = 8192
_COLS = 128
_BLOCK_ROWS = 2048


def _copy_block(t_ref, o_ref):
    o_ref[...] = t_ref[...]


def kernel(x, pos_table):
    del x
    n_blocks = _ROWS // _BLOCK_ROWS
    return pl.pallas_call(
        _copy_block,
        out_shape=jax.ShapeDtypeStruct((_ROWS, _COLS), pos_table.dtype),
        grid=(n_blocks,),
        in_specs=[pl.BlockSpec((_BLOCK_ROWS, _COLS), lambda i: (i, 0))],
        out_specs=pl.BlockSpec((_BLOCK_ROWS, _COLS), lambda i: (i, 0)),
    )(pos_table)


# speedup vs baseline: 33.8547x; 1.2985x over previous
import jax
import jax.numpy as jnp
from jax.experimental import pallas as pl

_ROWS = 8192
_COLS = 128
_BLOCK_ROWS = 4096


def _copy_block(t_ref, o_ref):
    o_ref[...] = t_ref[...]


def kernel(x, pos_table):
    del x
    n_blocks = _ROWS // _BLOCK_ROWS
    return pl.pallas_call(
        _copy_block,
        out_shape=jax.ShapeDtypeStruct((_ROWS, _COLS), pos_table.dtype),
        grid=(n_blocks,),
        in_specs=[pl.BlockSpec((_BLOCK_ROWS, _COLS), lambda i: (i, 0))],
        out_specs=pl.BlockSpec((_BLOCK_ROWS, _COLS), lambda i: (i, 0)),
    )(pos_table)
